# baseline (speedup 1.0000x reference)
import functools

import jax
import jax.numpy as jnp
from jax import lax
from jax.experimental import pallas as pl
from jax.experimental.pallas import tpu as pltpu
from jax.experimental.pallas import tpu_sc as plsc

N_NODES = 100000
BATCH = 16384
L = 16
NW = 32
RPW_MIN = 3120
RPW_PAD = 3136
LIST_LEN = RPW_PAD + L
G = 8


def _mesh():
    return plsc.VectorSubcoreMesh(
        core_axis_name="c", subcore_axis_name="s", num_cores=2, num_subcores=16
    )


def _worker_range():
    wid = lax.axis_index("c") * 16 + lax.axis_index("s")
    lo = (8 * ((wid * 12500) >> 5)).astype(jnp.int32)
    hi = (8 * (((wid + 1) * 12500) >> 5)).astype(jnp.int32)
    return wid, lo, hi


@functools.cache
def _build_prep():
    return pl.kernel(
        _prep_body,
        out_type=(
            jax.ShapeDtypeStruct((NW * RPW_PAD,), jnp.int32),
            jax.ShapeDtypeStruct((NW * RPW_PAD,), jnp.int32),
            jax.ShapeDtypeStruct((NW * L,), jnp.int32),
        ),
        mesh=_mesh(),
        compiler_params=pltpu.CompilerParams(needs_layout_passes=False),
        cost_estimate=pl.CostEstimate(
            flops=0, transcendentals=0, bytes_accessed=600_000_000),
        scratch_types=[
            pltpu.VMEM((BATCH,), jnp.int32),
            pltpu.VMEM((RPW_PAD,), jnp.int32),
            pltpu.VMEM((LIST_LEN,), jnp.int32),
            pltpu.VMEM((LIST_LEN,), jnp.int32),
            pltpu.VMEM((L,), jnp.int32),
        ],
    )


def _prep_body(idx_hbm, wl_hbm, rl_hbm, cnt_hbm, idx_v, aux, wlist, rlist,
               cnt_v):
    wid, lo, hi = _worker_range()
    lane = lax.iota(jnp.int32, L)

    pltpu.sync_copy(idx_hbm, idx_v)

    neg1 = jnp.full((L,), -1, jnp.int32)

    def init_body(i, carry):
        aux[pl.ds(i * L, L)] = neg1
        return carry

    lax.fori_loop(0, RPW_PAD // L, init_body, 0)

    def fill_body(i, carry):
        v = idx_v[pl.ds(i * L, L)]
        owned = (v >= lo) & (v < hi)
        _, last = plsc.scan_count(v, mask=owned)
        win = last & owned
        local = jnp.where(win, v - lo, 0)
        pos = (i * L + lane).astype(jnp.int32)
        plsc.store_scatter(aux, [local], pos, mask=win)
        return carry

    lax.fori_loop(0, BATCH // L, fill_body, 0)

    def comp_body(c, off):
        a = aux[pl.ds(c * L, L)]
        m = a >= 0
        rows = lo + c * L + lane
        plsc.store_compressed(wlist.at[pl.ds(off, L)], a, mask=m)
        plsc.store_compressed(rlist.at[pl.ds(off, L)], rows, mask=m)
        return off + jnp.sum(m.astype(jnp.int32))

    n = lax.fori_loop(0, RPW_PAD // L, comp_body, jnp.int32(0))

    cnt_v[...] = jnp.broadcast_to(n, (L,))
    base = wid * RPW_PAD
    pltpu.sync_copy(wlist.at[pl.ds(0, RPW_PAD)],
                    wl_hbm.at[pl.ds(base, RPW_PAD)])
    pltpu.sync_copy(rlist.at[pl.ds(0, RPW_PAD)],
                    rl_hbm.at[pl.ds(base, RPW_PAD)])
    pltpu.sync_copy(cnt_v, cnt_hbm.at[pl.ds(wid * L, L)])


@functools.cache
def _build_scatter():
    return pl.kernel(
        _scatter_body,
        out_type=(),
        mesh=_mesh(),
        compiler_params=pltpu.CompilerParams(needs_layout_passes=False),
        scratch_types=[
            pltpu.VMEM((RPW_PAD,), jnp.int32),
            pltpu.VMEM((RPW_PAD,), jnp.int32),
            pltpu.VMEM((L,), jnp.int32),
            pltpu.VMEM((G, L, 4, 128), jnp.float32),
            pltpu.SemaphoreType.DMA((G,)),
            pltpu.SemaphoreType.DMA((G,)),
        ],
    )


def _scatter_body(out, wl_hbm, rl_hbm, cnt_hbm, vals_hbm, wlist, rlist,
                  cnt_v, gbuf, gsem, ssem):
    wid, lo, hi = _worker_range()
    lane = lax.iota(jnp.int32, L)

    base = wid * RPW_PAD
    pltpu.sync_copy(wl_hbm.at[pl.ds(base, RPW_PAD)],
                    wlist.at[pl.ds(0, RPW_PAD)])
    pltpu.sync_copy(rl_hbm.at[pl.ds(base, RPW_PAD)],
                    rlist.at[pl.ds(0, RPW_PAD)])
    pltpu.sync_copy(cnt_hbm.at[pl.ds(wid * L, L)], cnt_v)
    n = jnp.max(cnt_v[...])

    @pl.when(n >= L)
    def _():
        nch = (n + L - 1) >> 4

        def grp_body(g, carry):
            gbase = g * G

            for j in range(G):
                @pl.when(gbase + j < nch)
                def _(j=j):
                    o = jnp.minimum((gbase + j) * L, n - L)
                    wv = wlist[pl.ds(o, L)]
                    pltpu.async_copy(vals_hbm.at[wv], gbuf.at[j], gsem.at[j])
            for j in range(G):
                @pl.when(gbase + j < nch)
                def _(j=j):
                    pltpu.make_async_copy(vals_hbm.at[wlist[pl.ds(0, L)]],
                                          gbuf.at[j], gsem.at[j]).wait()
                    o = jnp.minimum((gbase + j) * L, n - L)
                    rv = rlist[pl.ds(o, L)]
                    pltpu.async_copy(gbuf.at[j], out.at[rv], ssem.at[j])
            for j in range(G):
                @pl.when(gbase + j < nch)
                def _(j=j):
                    pltpu.make_async_copy(gbuf.at[j],
                                          out.at[rlist[pl.ds(0, L)]],
                                          ssem.at[j]).wait()
            return carry

        lax.fori_loop(0, (nch + G - 1) >> 3, grp_body, 0)

    @pl.when((n > 0) & (n < L))
    def _():
        wv = wlist[pl.ds(0, L)]
        rv = rlist[pl.ds(0, L)]

        def tail_body(i, carry):
            @pl.when(i < n)
            def _():
                wsc = jnp.max(jnp.where(lane == i, wv, -1))
                rsc = jnp.max(jnp.where(lane == i, rv, -1))
                pltpu.sync_copy(vals_hbm.at[pl.ds(wsc, 1)],
                                gbuf.at[0, pl.ds(0, 1)])
                pltpu.sync_copy(gbuf.at[0, pl.ds(0, 1)],
                                out.at[pl.ds(rsc, 1)])
            return carry

        lax.fori_loop(0, L, tail_body, 0)


def kernel(memory, node_idxs, values):
    idx = node_idxs.astype(jnp.int32)
    wl, rl, cnt = _build_prep()(idx)
    ref = jax.new_ref(memory)
    _build_scatter()(ref, wl, rl, cnt, values)
    return ref[...]


# speedup vs baseline: 1.0301x; 1.0301x over previous
import functools

import jax
import jax.numpy as jnp
from jax import lax
from jax.experimental import pallas as pl
from jax.experimental.pallas import tpu as pltpu
from jax.experimental.pallas import tpu_sc as plsc

N_NODES = 100000
BATCH = 16384
L = 16
NW = 32
RPW_MIN = 3120
RPW_PAD = 3136
LIST_LEN = RPW_PAD + L
CK = 48
NCH = RPW_MIN // CK
NS = 4
PRE = 2
SPC = 19
N_INIT = RPW_PAD // L
N_FILL = BATCH // L


@functools.cache
def _build_sc():
    mesh = plsc.VectorSubcoreMesh(
        core_axis_name="c", subcore_axis_name="s", num_cores=2, num_subcores=16
    )
    return pl.kernel(
        _body,
        out_type=jax.ShapeDtypeStruct((N_NODES, 4, 128), jnp.float32),
        mesh=mesh,
        compiler_params=pltpu.CompilerParams(needs_layout_passes=False),
        scratch_types=[
            pltpu.VMEM((BATCH,), jnp.int32),
            pltpu.VMEM((RPW_PAD,), jnp.int32),
            pltpu.VMEM((LIST_LEN,), jnp.int32),
            pltpu.VMEM((LIST_LEN,), jnp.int32),
            pltpu.VMEM((NS, CK, 4, 128), jnp.float32),
            pltpu.SemaphoreType.DMA((NS,)),
            pltpu.SemaphoreType.DMA((NS,)),
        ],
    )


def _body(mem, idx_hbm, vals_hbm, out, idx_v, aux, wlist, rlist, cbuf,
          gsem, ssem):
    wid = lax.axis_index("c") * 16 + lax.axis_index("s")
    lo = (8 * ((wid * 12500) >> 5)).astype(jnp.int32)
    hi = (8 * (((wid + 1) * 12500) >> 5)).astype(jnp.int32)
    lane = lax.iota(jnp.int32, L)
    neg1 = jnp.full((L,), -1, jnp.int32)

    for j in range(PRE):
        pltpu.async_copy(mem.at[pl.ds(lo + j * CK, CK)], cbuf.at[j],
                         gsem.at[j])
    pltpu.sync_copy(idx_hbm, idx_v)

    def scan_step(step):
        @pl.when(step < N_INIT)
        def _():
            aux[pl.ds(step * L, L)] = neg1

        @pl.when((step >= N_INIT) & (step < N_INIT + N_FILL))
        def _():
            i = step - N_INIT
            v = idx_v[pl.ds(i * L, L)]
            owned = (v >= lo) & (v < hi)
            _, last = plsc.scan_count(v, mask=owned)
            win = last & owned
            local = jnp.where(win, v - lo, 0)
            pos = (i * L + lane).astype(jnp.int32)
            plsc.store_scatter(aux, [local], pos, mask=win)

    def grp_body(g, carry):
        for j in range(NS):
            def chunk(j=j):
                k = g * NS + j

                def steps(t, c):
                    scan_step(k * SPC + t)
                    return c

                lax.fori_loop(0, SPC, steps, 0)

                kp = k + PRE
                jp = (j + PRE) % NS

                @pl.when(kp < NCH)
                def _():
                    @pl.when(kp - NS >= 0)
                    def _():
                        s_old = lo + (kp - NS) * CK
                        pltpu.make_async_copy(
                            cbuf.at[jp], out.at[pl.ds(s_old, CK)],
                            ssem.at[jp]).wait()
                    pltpu.async_copy(mem.at[pl.ds(lo + kp * CK, CK)],
                                     cbuf.at[jp], gsem.at[jp])

                s = lo + k * CK
                pltpu.make_async_copy(mem.at[pl.ds(s, CK)], cbuf.at[j],
                                      gsem.at[j]).wait()
                pltpu.async_copy(cbuf.at[j], out.at[pl.ds(s, CK)],
                                 ssem.at[j])

            @pl.when(g * NS + j < NCH)
            def _(chunk=chunk):
                chunk()
        return carry

    ngrp = (NCH + NS - 1) // NS
    lax.fori_loop(0, ngrp, grp_body, 0)

    for j in range(NS):
        k_last = NCH - NS + j
        jd = k_last % NS
        s_last = lo + k_last * CK
        pltpu.make_async_copy(cbuf.at[jd], out.at[pl.ds(s_last, CK)],
                              ssem.at[jd]).wait()

    total_steps = NCH * SPC
    need = N_INIT + N_FILL
    if total_steps < need:
        def rest(t, c):
            scan_step(t)
            return c
        lax.fori_loop(total_steps, need, rest, 0)

    @pl.when(hi - lo > RPW_MIN)
    def _():
        s = lo + RPW_MIN
        pltpu.sync_copy(mem.at[pl.ds(s, 8)], cbuf.at[0, pl.ds(0, 8)])
        pltpu.sync_copy(cbuf.at[0, pl.ds(0, 8)], out.at[pl.ds(s, 8)])

    def comp_body(c, off):
        a = aux[pl.ds(c * L, L)]
        m = a >= 0
        rows = lo + c * L + lane
        plsc.store_compressed(wlist.at[pl.ds(off, L)], a, mask=m)
        plsc.store_compressed(rlist.at[pl.ds(off, L)], rows, mask=m)
        return off + jnp.sum(m.astype(jnp.int32))

    n = lax.fori_loop(0, RPW_PAD // L, comp_body, jnp.int32(0))

    @pl.when(n >= L)
    def _():
        nch = (n + L - 1) >> 4

        def wgrp(g, carry):
            gbase = g * NS
            for j in range(NS):
                @pl.when(gbase + j < nch)
                def _(j=j):
                    o = jnp.minimum((gbase + j) * L, n - L)
                    wv = wlist[pl.ds(o, L)]
                    pltpu.async_copy(vals_hbm.at[wv],
                                     cbuf.at[j, pl.ds(0, L)], gsem.at[j])
            for j in range(NS):
                @pl.when(gbase + j < nch)
                def _(j=j):
                    pltpu.make_async_copy(vals_hbm.at[wlist[pl.ds(0, L)]],
                                          cbuf.at[j, pl.ds(0, L)],
                                          gsem.at[j]).wait()
                    o = jnp.minimum((gbase + j) * L, n - L)
                    rv = rlist[pl.ds(o, L)]
                    pltpu.async_copy(cbuf.at[j, pl.ds(0, L)], out.at[rv],
                                     ssem.at[j])
            for j in range(NS):
                @pl.when(gbase + j < nch)
                def _(j=j):
                    pltpu.make_async_copy(cbuf.at[j, pl.ds(0, L)],
                                          out.at[rlist[pl.ds(0, L)]],
                                          ssem.at[j]).wait()
            return carry

        lax.fori_loop(0, (nch + NS - 1) // NS, wgrp, 0)

    @pl.when((n > 0) & (n < L))
    def _():
        wv = wlist[pl.ds(0, L)]
        rv = rlist[pl.ds(0, L)]

        def tail_body(i, carry):
            @pl.when(i < n)
            def _():
                wsc = jnp.max(jnp.where(lane == i, wv, -1))
                rsc = jnp.max(jnp.where(lane == i, rv, -1))
                pltpu.sync_copy(vals_hbm.at[pl.ds(wsc, 1)],
                                cbuf.at[0, pl.ds(0, 1)])
                pltpu.sync_copy(cbuf.at[0, pl.ds(0, 1)],
                                out.at[pl.ds(rsc, 1)])
            return carry

        lax.fori_loop(0, L, tail_body, 0)


def kernel(memory, node_idxs, values):
    idx = node_idxs.astype(jnp.int32)
    return _build_sc()(memory, idx, values)


# speedup vs baseline: 1.0436x; 1.0132x over previous
import functools

import jax
import jax.numpy as jnp
from jax import lax
from jax.experimental import pallas as pl
from jax.experimental.pallas import tpu as pltpu
from jax.experimental.pallas import tpu_sc as plsc

N_NODES = 100000
BATCH = 16384
L = 16
NW = 32
RPW_MIN = 3120
RPW_PAD = 3136
LIST_LEN = RPW_PAD + L
CK = 24
NCH = RPW_MIN // CK
NS = 8
PRE = 4
SPC = 10
N_INIT = RPW_PAD // L
N_FILL = BATCH // L


@functools.cache
def _build_sc():
    mesh = plsc.VectorSubcoreMesh(
        core_axis_name="c", subcore_axis_name="s", num_cores=2, num_subcores=16
    )
    return pl.kernel(
        _body,
        out_type=jax.ShapeDtypeStruct((N_NODES, 4, 128), jnp.float32),
        mesh=mesh,
        compiler_params=pltpu.CompilerParams(needs_layout_passes=False),
        scratch_types=[
            pltpu.VMEM((BATCH,), jnp.int32),
            pltpu.VMEM((RPW_PAD,), jnp.int32),
            pltpu.VMEM((LIST_LEN,), jnp.int32),
            pltpu.VMEM((LIST_LEN,), jnp.int32),
            pltpu.VMEM((NS, CK, 4, 128), jnp.float32),
            pltpu.SemaphoreType.DMA((NS,)),
            pltpu.SemaphoreType.DMA((NS,)),
        ],
    )


def _body(mem, idx_hbm, vals_hbm, out, idx_v, aux, wlist, rlist, cbuf,
          gsem, ssem):
    wid = lax.axis_index("c") * 16 + lax.axis_index("s")
    lo = (8 * ((wid * 12500) >> 5)).astype(jnp.int32)
    hi = (8 * (((wid + 1) * 12500) >> 5)).astype(jnp.int32)
    lane = lax.iota(jnp.int32, L)
    neg1 = jnp.full((L,), -1, jnp.int32)

    for j in range(PRE):
        pltpu.async_copy(mem.at[pl.ds(lo + j * CK, CK)], cbuf.at[j],
                         gsem.at[j])
    pltpu.sync_copy(idx_hbm, idx_v)

    def scan_step(step):
        @pl.when(step < N_INIT)
        def _():
            aux[pl.ds(step * L, L)] = neg1

        @pl.when((step >= N_INIT) & (step < N_INIT + N_FILL))
        def _():
            i = step - N_INIT
            v = idx_v[pl.ds(i * L, L)]
            owned = (v >= lo) & (v < hi)
            _, last = plsc.scan_count(v, mask=owned)
            win = last & owned
            local = jnp.where(win, v - lo, 0)
            pos = (i * L + lane).astype(jnp.int32)
            plsc.store_scatter(aux, [local], pos, mask=win)

    def grp_body(g, carry):
        for j in range(NS):
            def chunk(j=j):
                k = g * NS + j

                def steps(t, c):
                    scan_step(k * SPC + t)
                    return c

                lax.fori_loop(0, SPC, steps, 0)

                kp = k + PRE
                jp = (j + PRE) % NS

                @pl.when(kp < NCH)
                def _():
                    @pl.when(kp - NS >= 0)
                    def _():
                        s_old = lo + (kp - NS) * CK
                        pltpu.make_async_copy(
                            cbuf.at[jp], out.at[pl.ds(s_old, CK)],
                            ssem.at[jp]).wait()
                    pltpu.async_copy(mem.at[pl.ds(lo + kp * CK, CK)],
                                     cbuf.at[jp], gsem.at[jp])

                s = lo + k * CK
                pltpu.make_async_copy(mem.at[pl.ds(s, CK)], cbuf.at[j],
                                      gsem.at[j]).wait()
                pltpu.async_copy(cbuf.at[j], out.at[pl.ds(s, CK)],
                                 ssem.at[j])

            @pl.when(g * NS + j < NCH)
            def _(chunk=chunk):
                chunk()
        return carry

    ngrp = (NCH + NS - 1) // NS
    lax.fori_loop(0, ngrp, grp_body, 0)

    for j in range(NS):
        k_last = NCH - NS + j
        jd = k_last % NS
        s_last = lo + k_last * CK
        pltpu.make_async_copy(cbuf.at[jd], out.at[pl.ds(s_last, CK)],
                              ssem.at[jd]).wait()

    total_steps = NCH * SPC
    need = N_INIT + N_FILL
    if total_steps < need:
        def rest(t, c):
            scan_step(t)
            return c
        lax.fori_loop(total_steps, need, rest, 0)

    @pl.when(hi - lo > RPW_MIN)
    def _():
        s = lo + RPW_MIN
        pltpu.sync_copy(mem.at[pl.ds(s, 8)], cbuf.at[0, pl.ds(0, 8)])
        pltpu.sync_copy(cbuf.at[0, pl.ds(0, 8)], out.at[pl.ds(s, 8)])

    def comp_body(c, off):
        a = aux[pl.ds(c * L, L)]
        m = a >= 0
        rows = lo + c * L + lane
        plsc.store_compressed(wlist.at[pl.ds(off, L)], a, mask=m)
        plsc.store_compressed(rlist.at[pl.ds(off, L)], rows, mask=m)
        return off + jnp.sum(m.astype(jnp.int32))

    n = lax.fori_loop(0, RPW_PAD // L, comp_body, jnp.int32(0))

    @pl.when(n >= L)
    def _():
        nch = (n + L - 1) >> 4

        def wgrp(g, carry):
            gbase = g * NS
            for j in range(NS):
                @pl.when(gbase + j < nch)
                def _(j=j):
                    o = jnp.minimum((gbase + j) * L, n - L)
                    wv = wlist[pl.ds(o, L)]
                    pltpu.async_copy(vals_hbm.at[wv],
                                     cbuf.at[j, pl.ds(0, L)], gsem.at[j])
            for j in range(NS):
                @pl.when(gbase + j < nch)
                def _(j=j):
                    pltpu.make_async_copy(vals_hbm.at[wlist[pl.ds(0, L)]],
                                          cbuf.at[j, pl.ds(0, L)],
                                          gsem.at[j]).wait()
                    o = jnp.minimum((gbase + j) * L, n - L)
                    rv = rlist[pl.ds(o, L)]
                    pltpu.async_copy(cbuf.at[j, pl.ds(0, L)], out.at[rv],
                                     ssem.at[j])
            for j in range(NS):
                @pl.when(gbase + j < nch)
                def _(j=j):
                    pltpu.make_async_copy(cbuf.at[j, pl.ds(0, L)],
                                          out.at[rlist[pl.ds(0, L)]],
                                          ssem.at[j]).wait()
            return carry

        lax.fori_loop(0, (nch + NS - 1) >> 3, wgrp, 0)

    @pl.when((n > 0) & (n < L))
    def _():
        wv = wlist[pl.ds(0, L)]
        rv = rlist[pl.ds(0, L)]

        def tail_body(i, carry):
            @pl.when(i < n)
            def _():
                wsc = jnp.max(jnp.where(lane == i, wv, -1))
                rsc = jnp.max(jnp.where(lane == i, rv, -1))
                pltpu.sync_copy(vals_hbm.at[pl.ds(wsc, 1)],
                                cbuf.at[0, pl.ds(0, 1)])
                pltpu.sync_copy(cbuf.at[0, pl.ds(0, 1)],
                                out.at[pl.ds(rsc, 1)])
            return carry

        lax.fori_loop(0, L, tail_body, 0)


def kernel(memory, node_idxs, values):
    idx = node_idxs.astype(jnp.int32)
    return _build_sc()(memory, idx, values)
